# baseline (speedup 1.0000x reference)
import functools

import jax
import jax.numpy as jnp
from jax import lax
from jax.experimental import pallas as pl
from jax.experimental.pallas import tpu as pltpu
from jax.experimental.pallas import tpu_sc as plsc

E_REAL = 8
E_TOT = 10
H = 2048
F = 1024
T = 8192
TM = 512
ROWS = 42 * TM
NT = ROWS // TM
RB = 512
TT = 512


def _router_body(x_ref, gwt_ref, logits_ref, i1_ref, i2_ref, w1_ref, w2_ref):
    x = x_ref[...]
    logits = jnp.dot(x, gwt_ref[...], preferred_element_type=jnp.float32)
    logits_ref[...] = logits
    m = jnp.max(logits, axis=1, keepdims=True)
    ex = jnp.exp(logits - m)
    p = ex / jnp.sum(ex, axis=1, keepdims=True)
    iota = lax.broadcasted_iota(jnp.int32, p.shape, 1)
    big = jnp.int32(1 << 30)
    m1 = jnp.max(p, axis=1, keepdims=True)
    i1 = jnp.min(jnp.where(p == m1, iota, big), axis=1, keepdims=True)
    pm = jnp.where(iota == i1, -jnp.inf, p)
    m2 = jnp.max(pm, axis=1, keepdims=True)
    i2 = jnp.min(jnp.where(pm == m2, iota, big), axis=1, keepdims=True)
    s = m1 + m2
    i1_ref[...] = i1
    i2_ref[...] = i2
    w1_ref[...] = m1 / s
    w2_ref[...] = m2 / s


def _router(x, gate_wt):
    return pl.pallas_call(
        _router_body,
        grid=(T // TT,),
        in_specs=[
            pl.BlockSpec((TT, H), lambda i: (i, 0)),
            pl.BlockSpec((H, E_TOT), lambda i: (0, 0)),
        ],
        out_specs=[
            pl.BlockSpec((TT, E_TOT), lambda i: (i, 0)),
            pl.BlockSpec((TT, 1), lambda i: (i, 0)),
            pl.BlockSpec((TT, 1), lambda i: (i, 0)),
            pl.BlockSpec((TT, 1), lambda i: (i, 0)),
            pl.BlockSpec((TT, 1), lambda i: (i, 0)),
        ],
        out_shape=[
            jax.ShapeDtypeStruct((T, E_TOT), jnp.float32),
            jax.ShapeDtypeStruct((T, 1), jnp.int32),
            jax.ShapeDtypeStruct((T, 1), jnp.int32),
            jax.ShapeDtypeStruct((T, 1), jnp.float32),
            jax.ShapeDtypeStruct((T, 1), jnp.float32),
        ],
    )(x, gate_wt)


def _rank_body(i1_ref, i2_ref, dst_ref, te_ref):
    lanes = lambda r: lax.broadcasted_iota(jnp.int32, (r, 16), 1)
    tri = (lax.broadcasted_iota(jnp.int32, (RB, RB), 0)
           > lax.broadcasted_iota(jnp.int32, (RB, RB), 1)).astype(jnp.float32)

    oh_all = ((i1_ref[...] == lanes(T)).astype(jnp.float32)
              + (i2_ref[...] == lanes(T)).astype(jnp.float32))
    c = jnp.sum(oh_all, axis=0, keepdims=True)
    pc = jnp.ceil(c / TM) * TM
    t16 = (lax.broadcasted_iota(jnp.int32, (16, 16), 0)
           < lax.broadcasted_iota(jnp.int32, (16, 16), 1)).astype(jnp.float32)
    off = jnp.dot(pc, t16, preferred_element_type=jnp.float32)

    def rank_body(ref, base):
        def body(b, carry):
            e = ref[pl.ds(b * RB, RB), :]
            oh = (e == lanes(RB)).astype(jnp.float32)
            exr = jnp.dot(tri, oh, preferred_element_type=jnp.float32) + carry
            dst = jnp.sum(oh * (exr + off), axis=1, keepdims=True)
            dst_ref[pl.ds(base + b * RB, RB), :] = dst.astype(jnp.int32)
            return carry + jnp.sum(oh, axis=0, keepdims=True)
        return body

    carry = jnp.zeros((1, 16), jnp.float32)
    carry = lax.fori_loop(0, T // RB, rank_body(i1_ref, 0), carry)
    carry = lax.fori_loop(0, T // RB, rank_body(i2_ref, T), carry)

    ends = off + pc
    starts = lax.broadcasted_iota(jnp.int32, (128, 16), 0).astype(jnp.float32) * TM
    te = jnp.sum((ends <= starts).astype(jnp.float32), axis=1, keepdims=True)
    te_ref[...] = te.astype(jnp.int32)


def _rank(i1, i2):
    return pl.pallas_call(
        _rank_body,
        grid=(1,),
        in_specs=[
            pl.BlockSpec((T, 1), lambda i: (0, 0)),
            pl.BlockSpec((T, 1), lambda i: (0, 0)),
        ],
        out_specs=[
            pl.BlockSpec((2 * T, 1), lambda i: (0, 0)),
            pl.BlockSpec((128, 1), lambda i: (0, 0)),
        ],
        out_shape=[
            jax.ShapeDtypeStruct((2 * T, 1), jnp.int32),
            jax.ShapeDtypeStruct((128, 1), jnp.int32),
        ],
    )(i1, i2)


SC_W = 16
SC_NW = 32


def _sc_dispatch(x, dstf):
    mesh = plsc.VectorSubcoreMesh(core_axis_name="c", subcore_axis_name="s")
    per_w = 2 * T // SC_NW
    n_chunks = per_w // SC_W

    @functools.partial(
        pl.kernel,
        out_type=jax.ShapeDtypeStruct((ROWS, H), jnp.float32),
        mesh=mesh,
        scratch_types=[
            pltpu.VMEM((2, SC_W), jnp.int32),
            pltpu.VMEM((2, SC_W, H), jnp.float32),
            pltpu.SemaphoreType.DMA,
            pltpu.SemaphoreType.DMA,
        ],
    )
    def k(x_hbm, d_hbm, xs_hbm, idx_v, rows_v, sem0, sem1):
        wid = lax.axis_index("s") * 2 + lax.axis_index("c")
        base = wid * per_w
        tbase = lax.rem(base, T)
        sems = (sem0, sem1)

        @pl.loop(0, n_chunks // 2)
        def _(g):
            for b in range(2):
                c = g * 2 + b

                @pl.when(g > 0)
                def _():
                    pltpu.make_async_copy(
                        rows_v.at[b], xs_hbm.at[idx_v.at[b]], sems[b]).wait()

                pltpu.sync_copy(d_hbm.at[pl.ds(base + c * SC_W, SC_W)],
                                idx_v.at[b])
                pltpu.sync_copy(x_hbm.at[pl.ds(tbase + c * SC_W, SC_W)],
                                rows_v.at[b])
                pltpu.async_copy(rows_v.at[b], xs_hbm.at[idx_v.at[b]], sems[b])

        for b in range(2):
            pltpu.make_async_copy(
                rows_v.at[b], xs_hbm.at[idx_v.at[b]], sems[b]).wait()

    return k(x, dstf)


def _mlp_body(te_ref, xs_ref, wgu_ref, wd_ref, out_ref):
    e = te_ref[pl.program_id(0)]

    @pl.when(e < E_REAL)
    def _():
        xs = xs_ref[...].astype(jnp.bfloat16)
        hu = jnp.dot(xs, wgu_ref[0], preferred_element_type=jnp.float32)
        h = hu[:, :F]
        u = hu[:, F:]
        act = ((h * lax.logistic(h)) * u).astype(jnp.bfloat16)
        out_ref[...] = jnp.dot(act, wd_ref[0], preferred_element_type=jnp.float32)

    @pl.when(e >= E_REAL)
    def _():
        out_ref[...] = jnp.zeros_like(out_ref)


def _clamped(te, m):
    return jnp.minimum(te[m], E_REAL - 1)


def _grouped_mlp(te, xs, wgu, w_down):
    return pl.pallas_call(
        _mlp_body,
        grid_spec=pltpu.PrefetchScalarGridSpec(
            num_scalar_prefetch=1,
            grid=(NT,),
            in_specs=[
                pl.BlockSpec((TM, H), lambda m, te: (m, 0)),
                pl.BlockSpec((1, H, 2 * F), lambda m, te: (_clamped(te, m), 0, 0)),
                pl.BlockSpec((1, F, H), lambda m, te: (_clamped(te, m), 0, 0)),
            ],
            out_specs=pl.BlockSpec((TM, H), lambda m, te: (m, 0)),
        ),
        out_shape=jax.ShapeDtypeStruct((ROWS, H), jnp.float32),
    )(te, xs, wgu, w_down)


def _sc_combine(rows, dstf):
    mesh = plsc.VectorSubcoreMesh(core_axis_name="c", subcore_axis_name="s")
    cw = SC_W // 2
    per_w = T // SC_NW
    n_chunks = per_w // cw

    @functools.partial(
        pl.kernel,
        out_type=[
            jax.ShapeDtypeStruct((T, H), jnp.float32),
            jax.ShapeDtypeStruct((T, H), jnp.float32),
        ],
        mesh=mesh,
        scratch_types=[
            pltpu.VMEM((2, cw), jnp.int32),
            pltpu.VMEM((2, cw), jnp.int32),
            pltpu.VMEM((2, cw, H), jnp.float32),
            pltpu.VMEM((2, cw, H), jnp.float32),
            pltpu.SemaphoreType.DMA,
            pltpu.SemaphoreType.DMA,
            pltpu.SemaphoreType.DMA,
            pltpu.SemaphoreType.DMA,
        ],
    )
    def k(rows_hbm, d_hbm, o0_hbm, o1_hbm, i0_v, i1_v, b0_v, b1_v,
          s00, s01, s10, s11):
        wid = lax.axis_index("s") * 2 + lax.axis_index("c")
        base = wid * per_w
        sems0 = (s00, s01)
        sems1 = (s10, s11)

        @pl.loop(0, n_chunks // 2)
        def _(g):
            for b in range(2):
                tb = base + (g * 2 + b) * cw

                @pl.when(g > 0)
                def _():
                    pltpu.make_async_copy(
                        b0_v.at[b], o0_hbm.at[pl.ds(tb, cw)], sems0[b]).wait()
                    pltpu.make_async_copy(
                        b1_v.at[b], o1_hbm.at[pl.ds(tb, cw)], sems1[b]).wait()

                pltpu.sync_copy(d_hbm.at[pl.ds(tb, cw)], i0_v.at[b])
                pltpu.sync_copy(rows_hbm.at[i0_v.at[b]], b0_v.at[b])
                pltpu.async_copy(b0_v.at[b], o0_hbm.at[pl.ds(tb, cw)],
                                 sems0[b])
                pltpu.sync_copy(d_hbm.at[pl.ds(T + tb, cw)], i1_v.at[b])
                pltpu.sync_copy(rows_hbm.at[i1_v.at[b]], b1_v.at[b])
                pltpu.async_copy(b1_v.at[b], o1_hbm.at[pl.ds(tb, cw)],
                                 sems1[b])

        for b in range(2):
            tb = base
            pltpu.make_async_copy(
                b0_v.at[b], o0_hbm.at[pl.ds(tb, cw)], sems0[b]).wait()
            pltpu.make_async_copy(
                b1_v.at[b], o1_hbm.at[pl.ds(tb, cw)], sems1[b]).wait()

    return k(rows, dstf)


def _combine_body(o0_ref, o1_ref, w1_ref, w2_ref, out_ref):
    out_ref[...] = w1_ref[...] * o0_ref[...] + w2_ref[...] * o1_ref[...]


def _combine(o0, o1, w1, w2):
    return pl.pallas_call(
        _combine_body,
        grid=(T // TT,),
        in_specs=[
            pl.BlockSpec((TT, H), lambda i: (i, 0)),
            pl.BlockSpec((TT, H), lambda i: (i, 0)),
            pl.BlockSpec((TT, 1), lambda i: (i, 0)),
            pl.BlockSpec((TT, 1), lambda i: (i, 0)),
        ],
        out_specs=pl.BlockSpec((TT, H), lambda i: (i, 0)),
        out_shape=jax.ShapeDtypeStruct((T, H), jnp.float32),
    )(o0, o1, w1, w2)


def kernel(hidden_states, gate_w, w_gate, w_up, w_down):
    b, s, d = hidden_states.shape
    x = hidden_states.reshape(-1, d)

    logits, i1, i2, w1, w2 = _router(x, gate_w.T)
    dst, te = _rank(i1, i2)

    dstf = dst.reshape(2 * T)
    xs = _sc_dispatch(x, dstf)
    wgu = jnp.concatenate([w_gate.astype(jnp.bfloat16),
                           w_up.astype(jnp.bfloat16)], axis=2)
    rows = _grouped_mlp(te[:NT, 0], xs, wgu, w_down.astype(jnp.bfloat16))
    o0, o1 = _sc_combine(rows, dstf)

    final = _combine(o0, o1, w1, w2)
    return final.reshape(b, s, d), logits


# speedup vs baseline: 1.0721x; 1.0721x over previous
import functools

import jax
import jax.numpy as jnp
from jax import lax
from jax.experimental import pallas as pl
from jax.experimental.pallas import tpu as pltpu
from jax.experimental.pallas import tpu_sc as plsc

E_REAL = 8
E_TOT = 10
H = 2048
F = 1024
T = 8192
TM = 512
ROWS = 42 * TM
NT = ROWS // TM
RB = 512
TT = 512


def _router_body(x_ref, gwt_ref, logits_ref, i1_ref, i2_ref, w1_ref, w2_ref):
    x = x_ref[...]
    logits = jnp.dot(x, gwt_ref[...], preferred_element_type=jnp.float32)
    logits_ref[...] = logits
    m = jnp.max(logits, axis=1, keepdims=True)
    ex = jnp.exp(logits - m)
    p = ex / jnp.sum(ex, axis=1, keepdims=True)
    iota = lax.broadcasted_iota(jnp.int32, p.shape, 1)
    big = jnp.int32(1 << 30)
    m1 = jnp.max(p, axis=1, keepdims=True)
    i1 = jnp.min(jnp.where(p == m1, iota, big), axis=1, keepdims=True)
    pm = jnp.where(iota == i1, -jnp.inf, p)
    m2 = jnp.max(pm, axis=1, keepdims=True)
    i2 = jnp.min(jnp.where(pm == m2, iota, big), axis=1, keepdims=True)
    s = m1 + m2
    i1_ref[...] = i1
    i2_ref[...] = i2
    w1_ref[...] = m1 / s
    w2_ref[...] = m2 / s


def _router(x, gate_wt):
    return pl.pallas_call(
        _router_body,
        grid=(T // TT,),
        in_specs=[
            pl.BlockSpec((TT, H), lambda i: (i, 0)),
            pl.BlockSpec((H, E_TOT), lambda i: (0, 0)),
        ],
        out_specs=[
            pl.BlockSpec((TT, E_TOT), lambda i: (i, 0)),
            pl.BlockSpec((TT, 1), lambda i: (i, 0)),
            pl.BlockSpec((TT, 1), lambda i: (i, 0)),
            pl.BlockSpec((TT, 1), lambda i: (i, 0)),
            pl.BlockSpec((TT, 1), lambda i: (i, 0)),
        ],
        out_shape=[
            jax.ShapeDtypeStruct((T, E_TOT), jnp.float32),
            jax.ShapeDtypeStruct((T, 1), jnp.int32),
            jax.ShapeDtypeStruct((T, 1), jnp.int32),
            jax.ShapeDtypeStruct((T, 1), jnp.float32),
            jax.ShapeDtypeStruct((T, 1), jnp.float32),
        ],
    )(x, gate_wt)


def _rank_body(i1_ref, i2_ref, dst_ref, te_ref):
    lanes = lambda r: lax.broadcasted_iota(jnp.int32, (r, 16), 1)
    tri = (lax.broadcasted_iota(jnp.int32, (RB, RB), 0)
           > lax.broadcasted_iota(jnp.int32, (RB, RB), 1)).astype(jnp.float32)

    oh_all = ((i1_ref[...] == lanes(T)).astype(jnp.float32)
              + (i2_ref[...] == lanes(T)).astype(jnp.float32))
    c = jnp.sum(oh_all, axis=0, keepdims=True)
    pc = jnp.ceil(c / TM) * TM
    t16 = (lax.broadcasted_iota(jnp.int32, (16, 16), 0)
           < lax.broadcasted_iota(jnp.int32, (16, 16), 1)).astype(jnp.float32)
    off = jnp.dot(pc, t16, preferred_element_type=jnp.float32)

    def rank_body(ref, base):
        def body(b, carry):
            e = ref[pl.ds(b * RB, RB), :]
            oh = (e == lanes(RB)).astype(jnp.float32)
            exr = jnp.dot(tri, oh, preferred_element_type=jnp.float32) + carry
            dst = jnp.sum(oh * (exr + off), axis=1, keepdims=True)
            dst_ref[pl.ds(base + b * RB, RB), :] = dst.astype(jnp.int32)
            return carry + jnp.sum(oh, axis=0, keepdims=True)
        return body

    carry = jnp.zeros((1, 16), jnp.float32)
    carry = lax.fori_loop(0, T // RB, rank_body(i1_ref, 0), carry)
    carry = lax.fori_loop(0, T // RB, rank_body(i2_ref, T), carry)

    ends = off + pc
    starts = lax.broadcasted_iota(jnp.int32, (128, 16), 0).astype(jnp.float32) * TM
    te = jnp.sum((ends <= starts).astype(jnp.float32), axis=1, keepdims=True)
    te_ref[...] = te.astype(jnp.int32)


def _rank(i1, i2):
    return pl.pallas_call(
        _rank_body,
        grid=(1,),
        in_specs=[
            pl.BlockSpec((T, 1), lambda i: (0, 0)),
            pl.BlockSpec((T, 1), lambda i: (0, 0)),
        ],
        out_specs=[
            pl.BlockSpec((2 * T, 1), lambda i: (0, 0)),
            pl.BlockSpec((128, 1), lambda i: (0, 0)),
        ],
        out_shape=[
            jax.ShapeDtypeStruct((2 * T, 1), jnp.int32),
            jax.ShapeDtypeStruct((128, 1), jnp.int32),
        ],
    )(i1, i2)


SC_W = 16
SC_NW = 32


def _sc_dispatch(x, dstf):
    mesh = plsc.VectorSubcoreMesh(core_axis_name="c", subcore_axis_name="s")
    per_w = 2 * T // SC_NW
    n_chunks = per_w // SC_W

    @functools.partial(
        pl.kernel,
        out_type=jax.ShapeDtypeStruct((ROWS, H), jnp.float32),
        mesh=mesh,
        scratch_types=[
            pltpu.VMEM((2, SC_W), jnp.int32),
            pltpu.VMEM((2, SC_W, H), jnp.float32),
            pltpu.SemaphoreType.DMA,
            pltpu.SemaphoreType.DMA,
        ],
    )
    def k(x_hbm, d_hbm, xs_hbm, idx_v, rows_v, sem0, sem1):
        wid = lax.axis_index("s") * 2 + lax.axis_index("c")
        base = wid * per_w
        tbase = lax.rem(base, T)
        sems = (sem0, sem1)

        @pl.loop(0, n_chunks // 2)
        def _(g):
            for b in range(2):
                c = g * 2 + b

                @pl.when(g > 0)
                def _():
                    pltpu.make_async_copy(
                        rows_v.at[b], xs_hbm.at[idx_v.at[b]], sems[b]).wait()

                pltpu.sync_copy(d_hbm.at[pl.ds(base + c * SC_W, SC_W)],
                                idx_v.at[b])
                pltpu.sync_copy(x_hbm.at[pl.ds(tbase + c * SC_W, SC_W)],
                                rows_v.at[b])
                pltpu.async_copy(rows_v.at[b], xs_hbm.at[idx_v.at[b]], sems[b])

        for b in range(2):
            pltpu.make_async_copy(
                rows_v.at[b], xs_hbm.at[idx_v.at[b]], sems[b]).wait()

    return k(x, dstf)


def _mlp_body(te_ref, xs_ref, wg_ref, wu_ref, wd_ref, out_ref):
    e = te_ref[pl.program_id(0)]

    @pl.when(e < E_REAL)
    def _():
        xs = xs_ref[...].astype(jnp.bfloat16)
        h = jnp.dot(xs, wg_ref[0], preferred_element_type=jnp.float32)
        u = jnp.dot(xs, wu_ref[0], preferred_element_type=jnp.float32)
        act = ((h * lax.logistic(h)) * u).astype(jnp.bfloat16)
        out_ref[...] = jnp.dot(act, wd_ref[0], preferred_element_type=jnp.float32)

    @pl.when(e >= E_REAL)
    def _():
        out_ref[...] = jnp.zeros_like(out_ref)


def _clamped(te, m):
    return jnp.minimum(te[m], E_REAL - 1)


def _grouped_mlp(te, xs, w_gate, w_up, w_down):
    return pl.pallas_call(
        _mlp_body,
        grid_spec=pltpu.PrefetchScalarGridSpec(
            num_scalar_prefetch=1,
            grid=(NT,),
            in_specs=[
                pl.BlockSpec((TM, H), lambda m, te: (m, 0)),
                pl.BlockSpec((1, H, F), lambda m, te: (_clamped(te, m), 0, 0)),
                pl.BlockSpec((1, H, F), lambda m, te: (_clamped(te, m), 0, 0)),
                pl.BlockSpec((1, F, H), lambda m, te: (_clamped(te, m), 0, 0)),
            ],
            out_specs=pl.BlockSpec((TM, H), lambda m, te: (m, 0)),
        ),
        out_shape=jax.ShapeDtypeStruct((ROWS, H), jnp.float32),
    )(te, xs, w_gate, w_up, w_down)


def _sc_combine(rows, dstf):
    mesh = plsc.VectorSubcoreMesh(core_axis_name="c", subcore_axis_name="s")
    per_w = T // SC_NW
    n_chunks = per_w // SC_W

    @functools.partial(
        pl.kernel,
        out_type=[
            jax.ShapeDtypeStruct((T, H), jnp.float32),
            jax.ShapeDtypeStruct((T, H), jnp.float32),
        ],
        mesh=mesh,
        scratch_types=[
            pltpu.VMEM((SC_W,), jnp.int32),
            pltpu.VMEM((SC_W, H), jnp.float32),
        ],
    )
    def k(rows_hbm, d_hbm, o0_hbm, o1_hbm, idx_v, buf_v):
        wid = lax.axis_index("s") * 2 + lax.axis_index("c")
        base = wid * per_w

        @pl.loop(0, n_chunks)
        def _(c):
            tb = base + c * SC_W
            pltpu.sync_copy(d_hbm.at[pl.ds(tb, SC_W)], idx_v)
            pltpu.sync_copy(rows_hbm.at[idx_v], buf_v)
            pltpu.sync_copy(buf_v, o0_hbm.at[pl.ds(tb, SC_W)])
            pltpu.sync_copy(d_hbm.at[pl.ds(T + tb, SC_W)], idx_v)
            pltpu.sync_copy(rows_hbm.at[idx_v], buf_v)
            pltpu.sync_copy(buf_v, o1_hbm.at[pl.ds(tb, SC_W)])

    return k(rows, dstf)


def _combine_body(o0_ref, o1_ref, w1_ref, w2_ref, out_ref):
    out_ref[...] = w1_ref[...] * o0_ref[...] + w2_ref[...] * o1_ref[...]


def _combine(o0, o1, w1, w2):
    return pl.pallas_call(
        _combine_body,
        grid=(T // TT,),
        in_specs=[
            pl.BlockSpec((TT, H), lambda i: (i, 0)),
            pl.BlockSpec((TT, H), lambda i: (i, 0)),
            pl.BlockSpec((TT, 1), lambda i: (i, 0)),
            pl.BlockSpec((TT, 1), lambda i: (i, 0)),
        ],
        out_specs=pl.BlockSpec((TT, H), lambda i: (i, 0)),
        out_shape=jax.ShapeDtypeStruct((T, H), jnp.float32),
    )(o0, o1, w1, w2)


def kernel(hidden_states, gate_w, w_gate, w_up, w_down):
    b, s, d = hidden_states.shape
    x = hidden_states.reshape(-1, d)

    logits, i1, i2, w1, w2 = _router(x, gate_w.T)
    dst, te = _rank(i1, i2)

    dstf = dst.reshape(2 * T)
    xs = _sc_dispatch(x, dstf)
    rows = _grouped_mlp(te[:NT, 0], xs,
                        w_gate.astype(jnp.bfloat16),
                        w_up.astype(jnp.bfloat16),
                        w_down.astype(jnp.bfloat16))
    o0, o1 = _sc_combine(rows, dstf)

    final = _combine(o0, o1, w1, w2)
    return final.reshape(b, s, d), logits
